# baseline (speedup 1.0000x reference)
import functools
import math

import jax
import jax.numpy as jnp
from jax import lax
from jax.experimental import pallas as pl
from jax.experimental.pallas import tpu as pltpu
from jax.experimental.pallas import tpu_sc as plsc

KERNEL_SIZE_FACTOR = 0.1
SIGMA = 3.0


def _kernel_consts(H):
    ks = int(H * KERNEL_SIZE_FACTOR)
    if ks % 2 == 0:
        ks += 1
    half = ks // 2
    c = (ks - 1) / 2.0
    z = sum(math.exp(-((i - c) ** 2) / (2.0 * SIGMA**2)) for i in range(ks))
    return ks, half, 1.0 / z


_NC = 2
_NS = 16
_L = 16


def _sc_scatter_counts(pts_t, B, P, H, W):
    MAP = H * W
    BPC = B // _NC
    TPB = _NS // BPC
    PER_TILE = P // TPB
    CORE_MAP = BPC * MAP
    SLICE = CORE_MAP // _NS
    ZCH = SLICE // 4

    mesh = plsc.VectorSubcoreMesh(
        core_axis_name="c", subcore_axis_name="s"
    )

    @functools.partial(
        pl.kernel,
        out_type=jax.ShapeDtypeStruct((B * MAP,), jnp.float32),
        mesh=mesh,
        scratch_types=[
            pltpu.VMEM((PER_TILE,), jnp.float32),
            pltpu.VMEM((PER_TILE,), jnp.float32),
            pltpu.VMEM((ZCH,), jnp.float32),
            pltpu.VMEM((_L,), jnp.float32),
            pltpu.VMEM_SHARED((CORE_MAP,), jnp.float32),
        ],
    )
    def sc_scatter(pts_hbm, out_hbm, xv, yv, zv, ones_v, smap):
        c = lax.axis_index("c")
        s = lax.axis_index("s")
        b_local = s // TPB
        b = c * BPC + b_local
        po = (s % TPB) * PER_TILE

        pltpu.sync_copy(pts_hbm.at[0, b, pl.ds(po, PER_TILE)], xv)
        pltpu.sync_copy(pts_hbm.at[1, b, pl.ds(po, PER_TILE)], yv)

        def zbody(i, carry):
            zv[pl.ds(i * _L, _L)] = jnp.zeros((_L,), jnp.float32)
            return carry

        lax.fori_loop(0, ZCH // _L, zbody, 0)
        ones_v[...] = jnp.ones((_L,), jnp.float32)
        base = s * SLICE
        for k in range(SLICE // ZCH):
            pltpu.sync_copy(zv, smap.at[pl.ds(base + k * ZCH, ZCH)])
        plsc.subcore_barrier()

        map_off = b_local * MAP

        def sbody(i, carry):
            x16 = xv[pl.ds(i * _L, _L)]
            y16 = yv[pl.ds(i * _L, _L)]
            xi = (x16 * W).astype(jnp.int32)
            yi = (y16 * H).astype(jnp.int32)
            idx = map_off + yi * W + xi
            pltpu.sync_copy(ones_v, smap.at[idx], add=True)
            return carry

        lax.fori_loop(0, PER_TILE // _L, sbody, 0)
        plsc.subcore_barrier()

        out_base = c * CORE_MAP + base
        pltpu.sync_copy(
            smap.at[pl.ds(base, SLICE)], out_hbm.at[pl.ds(out_base, SLICE)]
        )

    return sc_scatter(pts_t)


def _tc_blur_body(m_ref, o_ref, t_ref, *, H, half, inv_z):
    @pl.when(pl.program_id(0) == 0)
    def _():
        r = lax.broadcasted_iota(jnp.int32, (H, H), 0).astype(jnp.float32)
        cc = lax.broadcasted_iota(jnp.int32, (H, H), 1).astype(jnp.float32)
        d = r - cc
        inv_two_sigma2 = -1.0 / (2.0 * SIGMA * SIGMA)
        t_ref[...] = jnp.where(
            jnp.abs(d) <= half,
            jnp.exp(d * d * inv_two_sigma2) * inv_z,
            0.0,
        )

    T = t_ref[...]
    A = lax.dot_general(
        T, m_ref[0], (((1,), (0,)), ((), ())),
        preferred_element_type=jnp.float32,
    )
    o_ref[0] = lax.dot_general(
        A, T, (((1,), (0,)), ((), ())),
        preferred_element_type=jnp.float32,
    )


def _tc_blur(counts, B, H, W, half, inv_z):
    body = functools.partial(_tc_blur_body, H=H, half=half, inv_z=inv_z)
    return pl.pallas_call(
        body,
        grid=(B,),
        in_specs=[pl.BlockSpec((1, H, W), lambda b: (b, 0, 0))],
        out_specs=pl.BlockSpec((1, H, W), lambda b: (b, 0, 0)),
        out_shape=jax.ShapeDtypeStruct((B, H, W), jnp.float32),
        scratch_shapes=[pltpu.VMEM((H, H), jnp.float32)],
    )(counts)


def kernel(feature_map, points):
    B, C, H, W = feature_map.shape
    P = points.shape[1]
    ks, half, inv_z = _kernel_consts(min(H, W))

    pts_t = jnp.transpose(points, (2, 0, 1))

    counts = _sc_scatter_counts(pts_t, B, P, H, W).reshape(B, H, W)
    return _tc_blur(counts, B, H, W, half, inv_z)
